# baseline (speedup 1.0000x reference)
import functools

import jax
import jax.numpy as jnp
from jax.experimental import pallas as pl

_LN2 = 0.6931471805599453
_N_STREAMS = 4
_BLOCK_B = 2048


def _fused_log1p_matmul_kernel(*refs):
    x_refs = refs[:_N_STREAMS]
    emb_ref = refs[_N_STREAMS]
    out_ref = refs[_N_STREAMS + 1]
    feat = (emb_ref[...] * _LN2).astype(jnp.bfloat16)
    dims = (((1,), (0,)), ((), ()))
    for s in range(_N_STREAMS):
        y = jnp.log2(1.0 + x_refs[s][0]).astype(jnp.bfloat16)
        out_ref[s] = jax.lax.dot_general(
            y, feat, dimension_numbers=dims, preferred_element_type=jnp.float32
        )


@jax.jit
def kernel(x_seq, emb):
    B, P = x_seq.shape
    H = emb.shape[1]
    n_chunks = B // _BLOCK_B
    grid = (n_chunks // _N_STREAMS,)
    x3 = x_seq.reshape(n_chunks, _BLOCK_B, P)

    def x_spec(s):
        return pl.BlockSpec(
            (1, _BLOCK_B, P), lambda i, s=s: (_N_STREAMS * i + s, 0, 0)
        )

    out3 = pl.pallas_call(
        _fused_log1p_matmul_kernel,
        grid=grid,
        in_specs=[x_spec(s) for s in range(_N_STREAMS)]
        + [pl.BlockSpec((P, H), lambda i: (0, 0))],
        out_specs=pl.BlockSpec(
            (_N_STREAMS, _BLOCK_B, H), lambda i: (i, 0, 0)
        ),
        out_shape=jax.ShapeDtypeStruct((n_chunks, _BLOCK_B, H), jnp.float32),
    )(*([x3] * _N_STREAMS), emb)
    return out3.reshape(B, H)


# speedup vs baseline: 1.0695x; 1.0695x over previous
import functools

import jax
import jax.numpy as jnp
from jax.experimental import pallas as pl

_LN2 = 0.6931471805599453
_N_STREAMS = 8
_BLOCK_B = 512


def _fused_log1p_matmul_kernel(*refs):
    x_refs = refs[:_N_STREAMS]
    emb_ref = refs[_N_STREAMS]
    out_ref = refs[_N_STREAMS + 1]
    feat = (emb_ref[...] * _LN2).astype(jnp.bfloat16)
    dims = (((1,), (0,)), ((), ()))
    for s in range(_N_STREAMS):
        y = jnp.log2(1.0 + x_refs[s][0]).astype(jnp.bfloat16)
        out_ref[s] = jax.lax.dot_general(
            y, feat, dimension_numbers=dims, preferred_element_type=jnp.float32
        )


@jax.jit
def kernel(x_seq, emb):
    B, P = x_seq.shape
    H = emb.shape[1]
    n_chunks = B // _BLOCK_B
    grid = (n_chunks // _N_STREAMS,)
    x3 = x_seq.reshape(n_chunks, _BLOCK_B, P)

    def x_spec(s):
        return pl.BlockSpec(
            (1, _BLOCK_B, P), lambda i, s=s: (_N_STREAMS * i + s, 0, 0)
        )

    out3 = pl.pallas_call(
        _fused_log1p_matmul_kernel,
        grid=grid,
        in_specs=[x_spec(s) for s in range(_N_STREAMS)]
        + [pl.BlockSpec((P, H), lambda i: (0, 0))],
        out_specs=pl.BlockSpec(
            (_N_STREAMS, _BLOCK_B, H), lambda i: (i, 0, 0)
        ),
        out_shape=jax.ShapeDtypeStruct((n_chunks, _BLOCK_B, H), jnp.float32),
    )(*([x3] * _N_STREAMS), emb)
    return out3.reshape(B, H)
